# baseline (speedup 1.0000x reference)
import dataclasses
import functools

import jax
import jax.numpy as jnp
from jax import lax
from jax.experimental import pallas as pl
from jax.experimental.pallas import tpu as pltpu
from jax.experimental.pallas import tpu_sc as plsc

N = 10000
E = 320000
D = 128
DH = 64
DA = 80
NEG = 0.2
NC = 2
NS = 16
NW = NC * NS
EPW = E // NW
CH = 80
NCHUNK = EPW // CH
BCH = 25
NBLK = NCHUNK // BCH
ROWS_PER_TILE = N // NS

_SC_PARAMS = pltpu.CompilerParams(use_tc_tiling_on_sc=False)
if "needs_layout_passes" in pltpu.CompilerParams.__dataclass_fields__:
    _SC_PARAMS = dataclasses.replace(_SC_PARAMS, needs_layout_passes=False)


def _embed_body(x_ref, w_ref, att2_ref, ha_ref, hb_ref, as_ref, ad_ref):
    h = jnp.dot(x_ref[...], w_ref[...], preferred_element_type=jnp.float32)
    a2 = jnp.dot(h, att2_ref[...], preferred_element_type=jnp.float32)
    ha_ref[...] = jnp.concatenate(
        [h[:, :DH], a2[:, 0:1], jnp.zeros((N, DA - DH - 1), jnp.float32)],
        axis=1)
    hb_ref[...] = h[:, DH:]
    as_ref[...] = a2[:, 0:1]
    ad_ref[...] = a2[:, 1:2]


_tc_embed = pl.pallas_call(
    _embed_body,
    out_shape=(
        jax.ShapeDtypeStruct((N, DA), jnp.float32),
        jax.ShapeDtypeStruct((N, DH), jnp.float32),
        jax.ShapeDtypeStruct((N, 1), jnp.float32),
        jax.ShapeDtypeStruct((N, 1), jnp.float32),
    ),
)


def _zero_acc(rows, acc, sid, cols):
    zero16f = jnp.zeros((16,), jnp.float32)

    @pl.loop(0, CH)
    def _(i):
        for j in range(cols // 16):
            rows[i, pl.ds(j * 16, 16)] = zero16f

    base0 = sid * ROWS_PER_TILE

    @pl.loop(0, ROWS_PER_TILE // CH)
    def _(r):
        pltpu.sync_copy(rows, acc.at[pl.ds(base0 + r * CH, CH)])

    tail = ROWS_PER_TILE - (ROWS_PER_TILE // CH) * CH
    if tail:
        pltpu.sync_copy(rows.at[pl.ds(0, tail)],
                        acc.at[pl.ds(base0 + ROWS_PER_TILE - tail, tail)])


def _scale_rows(rows, wblk, c):
    rowc = jnp.full((16,), c, jnp.int32)
    for i in range(CH):
        wv = plsc.load_gather(wblk, [rowc, jnp.full((16,), i, jnp.int32)])
        row = rows.at[i]
        for r in range(DH // 16):
            row[pl.ds(r * 16, 16)] = row[pl.ds(r * 16, 16)] * wv


def _edge_pipeline(tbl_hbm, srcb, dstb, rows0, rows1, acc,
                   semg0, semg1, sems0, sems1, process):
    def wait_g(rows, sem):
        pltpu.make_async_copy(tbl_hbm.at[srcb.at[0]], rows, sem).wait()

    def wait_s(rows, sem):
        pltpu.make_async_copy(rows, acc.at[dstb.at[0]], sem).wait()

    pltpu.async_copy(tbl_hbm.at[srcb.at[0]], rows0, semg0)

    @pl.loop(0, (BCH - 1) // 2)
    def _(t):
        c0 = 2 * t
        wait_g(rows0, semg0)

        @pl.when(t > 0)
        def _():
            wait_s(rows1, sems1)

        pltpu.async_copy(tbl_hbm.at[srcb.at[c0 + 1]], rows1, semg1)
        process(rows0, c0)
        pltpu.async_copy(rows0, acc.at[dstb.at[c0]], sems0, add=True)
        wait_g(rows1, semg1)
        wait_s(rows0, sems0)
        pltpu.async_copy(tbl_hbm.at[srcb.at[c0 + 2]], rows0, semg0)
        process(rows1, c0 + 1)
        pltpu.async_copy(rows1, acc.at[dstb.at[c0 + 1]], sems1, add=True)

    wait_g(rows0, semg0)
    wait_s(rows1, sems1)
    process(rows0, BCH - 1)
    pltpu.async_copy(rows0, acc.at[dstb.at[BCH - 1]], sems0, add=True)
    wait_s(rows0, sems0)


def _sc_pass_a_body(adst_hbm, ha_hbm, src_hbm, dst_hbm, out_hbm, w_hbm,
                    srcb, dstb, adb, wblk, rows0, rows1,
                    acc, semg0, semg1, sems0, sems1, semad0, semad1, semw):
    cid = lax.axis_index("c")
    sid = lax.axis_index("s")
    wid = cid * NS + sid

    row_iota = lax.iota(jnp.int32, 16)
    col_as = jnp.full((16,), DH, jnp.int32)

    _zero_acc(rows0, acc, sid, DA)
    plsc.subcore_barrier()

    def compute_w(rows, c):
        for g in range(CH // 16):
            as16 = plsc.load_gather(rows, [row_iota + g * 16, col_as])
            e = as16 + adb[c, pl.ds(g * 16, 16)]
            e = jnp.where(e > 0.0, e, NEG * e)
            wblk[c, pl.ds(g * 16, 16)] = jnp.exp(e)

    def ad_issue(c, sem):
        pltpu.async_copy(adst_hbm.at[dstb.at[c]], adb.at[c], sem)

    def ad_wait(sem):
        pltpu.make_async_copy(adst_hbm.at[dstb.at[0]], adb.at[0], sem).wait()

    def process(rows, c, sem):
        ad_wait(sem)
        compute_w(rows, c)
        _scale_rows(rows, wblk, c)
        for g in range(CH // 16):
            w16 = wblk[c, pl.ds(g * 16, 16)]
            plsc.store_scatter(rows, [row_iota + g * 16, col_as], w16)

    @pl.loop(0, NBLK)
    def _(b):
        pltpu.sync_copy(src_hbm.at[wid, pl.ds(b * BCH, BCH)], srcb)
        pltpu.sync_copy(dst_hbm.at[wid, pl.ds(b * BCH, BCH)], dstb)

        @pl.when(b > 0)
        def _():
            pltpu.make_async_copy(
                wblk, w_hbm.at[wid, pl.ds(0, BCH)], semw).wait()

        ad_issue(0, semad0)
        ad_issue(1, semad1)

        def wait_g(rows, sem):
            pltpu.make_async_copy(ha_hbm.at[srcb.at[0]], rows, sem).wait()

        def wait_s(rows, sem):
            pltpu.make_async_copy(rows, acc.at[dstb.at[0]], sem).wait()

        pltpu.async_copy(ha_hbm.at[srcb.at[0]], rows0, semg0)

        @pl.loop(0, (BCH - 1) // 2)
        def _(t):
            c0 = 2 * t
            wait_g(rows0, semg0)

            @pl.when(t > 0)
            def _():
                wait_s(rows1, sems1)

            pltpu.async_copy(ha_hbm.at[srcb.at[c0 + 1]], rows1, semg1)
            process(rows0, c0, semad0)
            ad_issue(c0 + 2, semad0)
            pltpu.async_copy(rows0, acc.at[dstb.at[c0]], sems0, add=True)
            wait_g(rows1, semg1)
            wait_s(rows0, sems0)
            pltpu.async_copy(ha_hbm.at[srcb.at[c0 + 2]], rows0, semg0)
            process(rows1, c0 + 1, semad1)

            @pl.when(t < (BCH - 3) // 2)
            def _():
                ad_issue(c0 + 3, semad1)

            pltpu.async_copy(rows1, acc.at[dstb.at[c0 + 1]], sems1, add=True)

        wait_g(rows0, semg0)
        wait_s(rows1, sems1)
        process(rows0, BCH - 1, semad0)
        pltpu.async_copy(rows0, acc.at[dstb.at[BCH - 1]], sems0, add=True)
        wait_s(rows0, sems0)

        pltpu.async_copy(wblk, w_hbm.at[wid, pl.ds(b * BCH, BCH)], semw)

    pltpu.make_async_copy(wblk, w_hbm.at[wid, pl.ds(0, BCH)], semw).wait()
    plsc.subcore_barrier()

    base0 = sid * ROWS_PER_TILE
    pltpu.sync_copy(acc.at[pl.ds(base0, ROWS_PER_TILE)],
                    out_hbm.at[cid, pl.ds(base0, ROWS_PER_TILE)])


def _sc_pass_b_body(hb_hbm, src_hbm, dst_hbm, w_hbm, out_hbm,
                    srcb, dstb, wblk, r0, r1, acc,
                    sg0, sg1, ss0, ss1, ss2):
    cid = lax.axis_index("c")
    sid = lax.axis_index("s")
    wid = cid * NS + sid

    _zero_acc(r0, acc, sid, DH)
    plsc.subcore_barrier()

    def process(h):
        def body(rows, c):
            _scale_rows(rows, wblk.at[h], c)
        return body

    def blk_issue(b, h):
        pltpu.async_copy(src_hbm.at[wid, pl.ds(b * BCH, BCH)],
                         srcb.at[h], ss2)
        pltpu.async_copy(dst_hbm.at[wid, pl.ds(b * BCH, BCH)],
                         dstb.at[h], ss2)
        pltpu.async_copy(w_hbm.at[wid, pl.ds(b * BCH, BCH)], wblk.at[h], ss2)

    def blk_wait(h):
        pltpu.make_async_copy(src_hbm.at[wid, pl.ds(0, BCH)],
                              srcb.at[h], ss2).wait()
        pltpu.make_async_copy(dst_hbm.at[wid, pl.ds(0, BCH)],
                              dstb.at[h], ss2).wait()
        pltpu.make_async_copy(w_hbm.at[wid, pl.ds(0, BCH)],
                              wblk.at[h], ss2).wait()

    def run(h):
        _edge_pipeline(hb_hbm, srcb.at[h], dstb.at[h], r0, r1, acc,
                       sg0, sg1, ss0, ss1, process(h))

    blk_issue(0, 0)

    @pl.loop(0, (NBLK - 1) // 2)
    def _(t):
        blk_wait(0)
        blk_issue(2 * t + 1, 1)
        run(0)
        blk_wait(1)
        blk_issue(2 * t + 2, 0)
        run(1)

    blk_wait(0)
    run(0)

    plsc.subcore_barrier()

    base0 = sid * ROWS_PER_TILE
    pltpu.sync_copy(acc.at[pl.ds(base0, ROWS_PER_TILE)],
                    out_hbm.at[cid, pl.ds(base0, ROWS_PER_TILE)])


_MESH = dict(
    mesh=plsc.VectorSubcoreMesh(core_axis_name="c", subcore_axis_name="s"),
    compiler_params=_SC_PARAMS,
)

_sc_pass_a = pl.kernel(
    _sc_pass_a_body,
    out_type=(
        jax.ShapeDtypeStruct((NC, N, DA), jnp.float32),
        jax.ShapeDtypeStruct((NW, NCHUNK, CH), jnp.float32),
    ),
    scratch_types=[
        pltpu.VMEM((BCH, CH), jnp.int32),
        pltpu.VMEM((BCH, CH), jnp.int32),
        pltpu.VMEM((BCH, CH), jnp.float32),
        pltpu.VMEM((BCH, CH), jnp.float32),
        pltpu.VMEM((CH, DA), jnp.float32),
        pltpu.VMEM((CH, DA), jnp.float32),
        pltpu.VMEM_SHARED((N, DA), jnp.float32),
        pltpu.SemaphoreType.DMA,
        pltpu.SemaphoreType.DMA,
        pltpu.SemaphoreType.DMA,
        pltpu.SemaphoreType.DMA,
        pltpu.SemaphoreType.DMA,
        pltpu.SemaphoreType.DMA,
        pltpu.SemaphoreType.DMA,
    ],
    **_MESH,
)

_sc_pass_b = pl.kernel(
    _sc_pass_b_body,
    out_type=jax.ShapeDtypeStruct((NC, N, DH), jnp.float32),
    scratch_types=[
        pltpu.VMEM((2, BCH, CH), jnp.int32),
        pltpu.VMEM((2, BCH, CH), jnp.int32),
        pltpu.VMEM((2, BCH, CH), jnp.float32),
        pltpu.VMEM((CH, DH), jnp.float32),
        pltpu.VMEM((CH, DH), jnp.float32),
        pltpu.VMEM_SHARED((N, DH), jnp.float32),
        pltpu.SemaphoreType.DMA,
        pltpu.SemaphoreType.DMA,
        pltpu.SemaphoreType.DMA,
        pltpu.SemaphoreType.DMA,
        pltpu.SemaphoreType.DMA,
    ],
    **_MESH,
)


def _combine_body(pa_ref, pb_ref, ha_ref, hb_ref, as_ref, ad_ref, b_ref,
                  o_ref):
    pa0 = pa_ref[0]
    pa1 = pa_ref[1]
    pb0 = pb_ref[0]
    pb1 = pb_ref[1]
    num = jnp.concatenate(
        [pa0[:, :DH] + pa1[:, :DH], pb0 + pb1], axis=1)
    den = pa0[:, DH:DH + 1] + pa1[:, DH:DH + 1]
    es = as_ref[...] + ad_ref[...]
    es = jnp.where(es > 0.0, es, NEG * es)
    ws = jnp.exp(es)
    h = jnp.concatenate([ha_ref[:, :DH], hb_ref[...]], axis=1)
    o_ref[...] = (num + ws * h) / (den + ws + 1e-16) + b_ref[...]


_CB = 1000

_tc_combine = pl.pallas_call(
    _combine_body,
    grid=(N // _CB,),
    in_specs=[
        pl.BlockSpec((NC, _CB, DA), lambda i: (0, i, 0)),
        pl.BlockSpec((NC, _CB, DH), lambda i: (0, i, 0)),
        pl.BlockSpec((_CB, DA), lambda i: (i, 0)),
        pl.BlockSpec((_CB, DH), lambda i: (i, 0)),
        pl.BlockSpec((_CB, 1), lambda i: (i, 0)),
        pl.BlockSpec((_CB, 1), lambda i: (i, 0)),
        pl.BlockSpec((1, D), lambda i: (0, 0)),
    ],
    out_specs=pl.BlockSpec((_CB, D), lambda i: (i, 0)),
    out_shape=jax.ShapeDtypeStruct((N, D), jnp.float32),
)


def kernel(x, edge_index, W, att_src, att_dst, bias):
    att2 = jnp.stack([att_src, att_dst], axis=1)
    src = edge_index[0].astype(jnp.int32).reshape(NW, NCHUNK, CH)
    dst = edge_index[1].astype(jnp.int32).reshape(NW, NCHUNK, CH)
    h_a, h_b, a_s, a_d = _tc_embed(x, W, att2)
    adf = a_d.reshape(N)
    pa, wsaved = _sc_pass_a(adf, h_a, src, dst)
    pb = _sc_pass_b(h_b, src, dst, wsaved)
    return _tc_combine(pa, pb, h_a, h_b, a_s, a_d, bias.reshape(1, D))


# speedup vs baseline: 1.0738x; 1.0738x over previous
import dataclasses
import functools

import jax
import jax.numpy as jnp
from jax import lax
from jax.experimental import pallas as pl
from jax.experimental.pallas import tpu as pltpu
from jax.experimental.pallas import tpu_sc as plsc

N = 10000
E = 320000
D = 128
DH = 64
DA = 80
NEG = 0.2
NC = 2
NS = 16
NW = NC * NS
EPW = E // NW
CH = 80
NCHUNK = EPW // CH
BCH = 25
NBLK = NCHUNK // BCH
ROWS_PER_TILE = N // NS

_SC_PARAMS = pltpu.CompilerParams(use_tc_tiling_on_sc=False)
if "needs_layout_passes" in pltpu.CompilerParams.__dataclass_fields__:
    _SC_PARAMS = dataclasses.replace(_SC_PARAMS, needs_layout_passes=False)


def _embed_body(x_ref, w_ref, att2_ref, ha_ref, hb_ref, as_ref, ad_ref):
    h = jnp.dot(x_ref[...], w_ref[...], preferred_element_type=jnp.float32)
    a2 = jnp.dot(h, att2_ref[...], preferred_element_type=jnp.float32)
    ha_ref[...] = jnp.concatenate(
        [h[:, :DH], a2[:, 0:1], jnp.zeros((N, DA - DH - 1), jnp.float32)],
        axis=1)
    hb_ref[...] = h[:, DH:]
    as_ref[...] = a2[:, 0:1]
    ad_ref[...] = a2[:, 1:2]


_tc_embed = pl.pallas_call(
    _embed_body,
    out_shape=(
        jax.ShapeDtypeStruct((N, DA), jnp.float32),
        jax.ShapeDtypeStruct((N, DH), jnp.float32),
        jax.ShapeDtypeStruct((N, 1), jnp.float32),
        jax.ShapeDtypeStruct((N, 1), jnp.float32),
    ),
)


def _zero_acc(rows, acc, sid, cols):
    zero16f = jnp.zeros((16,), jnp.float32)

    @pl.loop(0, CH)
    def _(i):
        for j in range(cols // 16):
            rows[i, pl.ds(j * 16, 16)] = zero16f

    base0 = sid * ROWS_PER_TILE

    @pl.loop(0, ROWS_PER_TILE // CH)
    def _(r):
        pltpu.sync_copy(rows, acc.at[pl.ds(base0 + r * CH, CH)])

    tail = ROWS_PER_TILE - (ROWS_PER_TILE // CH) * CH
    if tail:
        pltpu.sync_copy(rows.at[pl.ds(0, tail)],
                        acc.at[pl.ds(base0 + ROWS_PER_TILE - tail, tail)])


def _scale_rows(rows, wblk, c):
    rowc = jnp.full((16,), c, jnp.int32)
    for i in range(CH):
        wv = plsc.load_gather(wblk, [rowc, jnp.full((16,), i, jnp.int32)])
        row = rows.at[i]
        for r in range(DH // 16):
            row[pl.ds(r * 16, 16)] = row[pl.ds(r * 16, 16)] * wv


def _edge_pipeline(tbl_hbm, srcb, dstb, rows0, rows1, acc,
                   semg0, semg1, sems0, sems1, process):
    def wait_g(rows, sem):
        pltpu.make_async_copy(tbl_hbm.at[srcb.at[0]], rows, sem).wait()

    def wait_s(rows, sem):
        pltpu.make_async_copy(rows, acc.at[dstb.at[0]], sem).wait()

    pltpu.async_copy(tbl_hbm.at[srcb.at[0]], rows0, semg0)

    @pl.loop(0, (BCH - 1) // 2)
    def _(t):
        c0 = 2 * t
        wait_g(rows0, semg0)

        @pl.when(t > 0)
        def _():
            wait_s(rows1, sems1)

        pltpu.async_copy(tbl_hbm.at[srcb.at[c0 + 1]], rows1, semg1)
        process(rows0, c0)
        pltpu.async_copy(rows0, acc.at[dstb.at[c0]], sems0, add=True)
        wait_g(rows1, semg1)
        wait_s(rows0, sems0)
        pltpu.async_copy(tbl_hbm.at[srcb.at[c0 + 2]], rows0, semg0)
        process(rows1, c0 + 1)
        pltpu.async_copy(rows1, acc.at[dstb.at[c0 + 1]], sems1, add=True)

    wait_g(rows0, semg0)
    wait_s(rows1, sems1)
    process(rows0, BCH - 1)
    pltpu.async_copy(rows0, acc.at[dstb.at[BCH - 1]], sems0, add=True)
    wait_s(rows0, sems0)


def _sc_pass_a_body(adst_hbm, ha_hbm, src_hbm, dst_hbm, out_hbm, w_hbm,
                    srcb, dstb, adb, wblk, rows0, rows1,
                    acc, semg0, semg1, sems0, sems1, semad0, semad1, semw):
    cid = lax.axis_index("c")
    sid = lax.axis_index("s")
    wid = cid * NS + sid

    row_iota = lax.iota(jnp.int32, 16)
    col_as = jnp.full((16,), DH, jnp.int32)

    _zero_acc(rows0, acc, sid, DA)
    plsc.subcore_barrier()

    def compute_w(rows, c):
        for g in range(CH // 16):
            as16 = plsc.load_gather(rows, [row_iota + g * 16, col_as])
            e = as16 + adb[c, pl.ds(g * 16, 16)]
            e = jnp.where(e > 0.0, e, NEG * e)
            wblk[c, pl.ds(g * 16, 16)] = jnp.exp(e)

    def ad_issue(c, sem):
        pltpu.async_copy(adst_hbm.at[dstb.at[c]], adb.at[c], sem)

    def ad_wait(sem):
        pltpu.make_async_copy(adst_hbm.at[dstb.at[0]], adb.at[0], sem).wait()

    def process(rows, c, sem):
        ad_wait(sem)
        compute_w(rows, c)
        _scale_rows(rows, wblk, c)
        for g in range(CH // 16):
            w16 = wblk[c, pl.ds(g * 16, 16)]
            plsc.store_scatter(rows, [row_iota + g * 16, col_as], w16)

    @pl.loop(0, NBLK)
    def _(b):
        pltpu.sync_copy(src_hbm.at[wid, pl.ds(b * BCH, BCH)], srcb)
        pltpu.sync_copy(dst_hbm.at[wid, pl.ds(b * BCH, BCH)], dstb)

        @pl.when(b > 0)
        def _():
            pltpu.make_async_copy(
                wblk, w_hbm.at[wid, pl.ds(0, BCH)], semw).wait()

        ad_issue(0, semad0)
        ad_issue(1, semad1)

        def wait_g(rows, sem):
            pltpu.make_async_copy(ha_hbm.at[srcb.at[0]], rows, sem).wait()

        def wait_s(rows, sem):
            pltpu.make_async_copy(rows, acc.at[dstb.at[0]], sem).wait()

        pltpu.async_copy(ha_hbm.at[srcb.at[0]], rows0, semg0)

        @pl.loop(0, (BCH - 1) // 2)
        def _(t):
            c0 = 2 * t
            wait_g(rows0, semg0)

            @pl.when(t > 0)
            def _():
                wait_s(rows1, sems1)

            pltpu.async_copy(ha_hbm.at[srcb.at[c0 + 1]], rows1, semg1)
            process(rows0, c0, semad0)
            ad_issue(c0 + 2, semad0)
            pltpu.async_copy(rows0, acc.at[dstb.at[c0]], sems0, add=True)
            wait_g(rows1, semg1)
            wait_s(rows0, sems0)
            pltpu.async_copy(ha_hbm.at[srcb.at[c0 + 2]], rows0, semg0)
            process(rows1, c0 + 1, semad1)

            @pl.when(t < (BCH - 3) // 2)
            def _():
                ad_issue(c0 + 3, semad1)

            pltpu.async_copy(rows1, acc.at[dstb.at[c0 + 1]], sems1, add=True)

        wait_g(rows0, semg0)
        wait_s(rows1, sems1)
        process(rows0, BCH - 1, semad0)
        pltpu.async_copy(rows0, acc.at[dstb.at[BCH - 1]], sems0, add=True)
        wait_s(rows0, sems0)

        pltpu.async_copy(wblk, w_hbm.at[wid, pl.ds(b * BCH, BCH)], semw)

    pltpu.make_async_copy(wblk, w_hbm.at[wid, pl.ds(0, BCH)], semw).wait()
    plsc.subcore_barrier()

    base0 = sid * ROWS_PER_TILE
    pltpu.sync_copy(acc.at[pl.ds(base0, ROWS_PER_TILE)],
                    out_hbm.at[cid, pl.ds(base0, ROWS_PER_TILE)])


def _sc_pass_b_body(hb_hbm, src_hbm, dst_hbm, w_hbm, out_hbm,
                    srcb, dstb, wblk, r0, r1, acc,
                    sg0, sg1, ss0, ss1, ss2):
    cid = lax.axis_index("c")
    sid = lax.axis_index("s")
    wid = cid * NS + sid
    del ss2

    _zero_acc(r0, acc, sid, DH)
    plsc.subcore_barrier()

    def process(rows, c):
        _scale_rows(rows, wblk, c)

    @pl.loop(0, NBLK)
    def _(b):
        pltpu.sync_copy(src_hbm.at[wid, pl.ds(b * BCH, BCH)], srcb)
        pltpu.sync_copy(dst_hbm.at[wid, pl.ds(b * BCH, BCH)], dstb)
        pltpu.sync_copy(w_hbm.at[wid, pl.ds(b * BCH, BCH)], wblk)

        _edge_pipeline(hb_hbm, srcb, dstb, r0, r1, acc,
                       sg0, sg1, ss0, ss1, process)

    plsc.subcore_barrier()

    base0 = sid * ROWS_PER_TILE
    pltpu.sync_copy(acc.at[pl.ds(base0, ROWS_PER_TILE)],
                    out_hbm.at[cid, pl.ds(base0, ROWS_PER_TILE)])


_MESH = dict(
    mesh=plsc.VectorSubcoreMesh(core_axis_name="c", subcore_axis_name="s"),
    compiler_params=_SC_PARAMS,
)

_sc_pass_a = pl.kernel(
    _sc_pass_a_body,
    out_type=(
        jax.ShapeDtypeStruct((NC, N, DA), jnp.float32),
        jax.ShapeDtypeStruct((NW, NCHUNK, CH), jnp.float32),
    ),
    scratch_types=[
        pltpu.VMEM((BCH, CH), jnp.int32),
        pltpu.VMEM((BCH, CH), jnp.int32),
        pltpu.VMEM((BCH, CH), jnp.float32),
        pltpu.VMEM((BCH, CH), jnp.float32),
        pltpu.VMEM((CH, DA), jnp.float32),
        pltpu.VMEM((CH, DA), jnp.float32),
        pltpu.VMEM_SHARED((N, DA), jnp.float32),
        pltpu.SemaphoreType.DMA,
        pltpu.SemaphoreType.DMA,
        pltpu.SemaphoreType.DMA,
        pltpu.SemaphoreType.DMA,
        pltpu.SemaphoreType.DMA,
        pltpu.SemaphoreType.DMA,
        pltpu.SemaphoreType.DMA,
    ],
    **_MESH,
)

_sc_pass_b = pl.kernel(
    _sc_pass_b_body,
    out_type=jax.ShapeDtypeStruct((NC, N, DH), jnp.float32),
    scratch_types=[
        pltpu.VMEM((BCH, CH), jnp.int32),
        pltpu.VMEM((BCH, CH), jnp.int32),
        pltpu.VMEM((BCH, CH), jnp.float32),
        pltpu.VMEM((CH, DH), jnp.float32),
        pltpu.VMEM((CH, DH), jnp.float32),
        pltpu.VMEM_SHARED((N, DH), jnp.float32),
        pltpu.SemaphoreType.DMA,
        pltpu.SemaphoreType.DMA,
        pltpu.SemaphoreType.DMA,
        pltpu.SemaphoreType.DMA,
        pltpu.SemaphoreType.DMA,
    ],
    **_MESH,
)


def _combine_body(pa_ref, pb_ref, ha_ref, hb_ref, as_ref, ad_ref, b_ref,
                  o_ref):
    pa0 = pa_ref[0]
    pa1 = pa_ref[1]
    pb0 = pb_ref[0]
    pb1 = pb_ref[1]
    num = jnp.concatenate(
        [pa0[:, :DH] + pa1[:, :DH], pb0 + pb1], axis=1)
    den = pa0[:, DH:DH + 1] + pa1[:, DH:DH + 1]
    es = as_ref[...] + ad_ref[...]
    es = jnp.where(es > 0.0, es, NEG * es)
    ws = jnp.exp(es)
    h = jnp.concatenate([ha_ref[:, :DH], hb_ref[...]], axis=1)
    o_ref[...] = (num + ws * h) / (den + ws + 1e-16) + b_ref[...]


_CB = 1000

_tc_combine = pl.pallas_call(
    _combine_body,
    grid=(N // _CB,),
    in_specs=[
        pl.BlockSpec((NC, _CB, DA), lambda i: (0, i, 0)),
        pl.BlockSpec((NC, _CB, DH), lambda i: (0, i, 0)),
        pl.BlockSpec((_CB, DA), lambda i: (i, 0)),
        pl.BlockSpec((_CB, DH), lambda i: (i, 0)),
        pl.BlockSpec((_CB, 1), lambda i: (i, 0)),
        pl.BlockSpec((_CB, 1), lambda i: (i, 0)),
        pl.BlockSpec((1, D), lambda i: (0, 0)),
    ],
    out_specs=pl.BlockSpec((_CB, D), lambda i: (i, 0)),
    out_shape=jax.ShapeDtypeStruct((N, D), jnp.float32),
)


def kernel(x, edge_index, W, att_src, att_dst, bias):
    att2 = jnp.stack([att_src, att_dst], axis=1)
    src = edge_index[0].astype(jnp.int32).reshape(NW, NCHUNK, CH)
    dst = edge_index[1].astype(jnp.int32).reshape(NW, NCHUNK, CH)
    h_a, h_b, a_s, a_d = _tc_embed(x, W, att2)
    adf = a_d.reshape(N)
    pa, wsaved = _sc_pass_a(adf, h_a, src, dst)
    pb = _sc_pass_b(h_b, src, dst, wsaved)
    return _tc_combine(pa, pb, h_a, h_b, a_s, a_d, bias.reshape(1, D))
